# baseline (speedup 1.0000x reference)
import functools

import numpy as np

import jax
import jax.numpy as jnp
from jax import lax
from jax.experimental import pallas as pl
from jax.experimental.pallas import tpu as pltpu
from jax.experimental.pallas import tpu_sc as plsc

N = 10000
D = 128
E = 320000
NC = 2
NS = 16
NW = NC * NS
C = 96
G = 21
NG0 = 5
NG1 = 5
GEDGES = G * C
NGTOT = NS * (NG0 + NG1)
EPAD = NGTOT * GEDGES
RPT = 632
NPAD = NS * RPT
NBUF = 3


def _make_shift():
  mesh = plsc.VectorSubcoreMesh(
      core_axis_name="c", subcore_axis_name="s", num_cores=NC, num_subcores=NS)
  out_type = (
      jax.ShapeDtypeStruct((NPAD, D), jnp.float32),
      jax.ShapeDtypeStruct((NPAD, D), jnp.float32),
  )
  scratch = (
      [
          pltpu.VMEM((G, C), jnp.int32),
          pltpu.VMEM((G, C), jnp.int32),
          pltpu.VMEM((G * C + 16,), jnp.float32),
      ]
      + [pltpu.VMEM((C, D), jnp.float32)] * NBUF
      + [pltpu.VMEM_SHARED((NPAD, D), jnp.float32)]
      + [pltpu.SemaphoreType.DMA] * (2 * NBUF)
  )

  @functools.partial(
      pl.kernel, out_type=out_type, mesh=mesh, scratch_types=scratch)
  def shift(h0, srch, dsth, attrh, o0, o1, srcv, dstv, attrv, *bufs_sems):
    gbuf = bufs_sems[:NBUF]
    acc = bufs_sems[NBUF]
    gsem = bufs_sems[NBUF + 1:NBUF + 1 + NBUF]
    ssem = bufs_sems[NBUF + 1 + NBUF:]
    cid = lax.axis_index("c")
    sid = lax.axis_index("s")
    wid = sid * NC + cid

    zero = jnp.zeros((16,), jnp.float32)

    def zrow(r, _):
      for u in range(D // 16):
        gbuf[0][r, pl.ds(u * 16, 16)] = zero
      return 0

    lax.fori_loop(0, C, zrow, 0)
    base = sid * RPT
    nfull, rem = divmod(RPT, C)
    for t in range(nfull):
      pltpu.sync_copy(gbuf[0], acc.at[pl.ds(base + t * C, C)])
    if rem:
      pltpu.sync_copy(gbuf[0].at[pl.ds(0, rem)],
                      acc.at[pl.ds(base + nfull * C, rem)])
    plsc.subcore_barrier()

    def scale(j, b):
      def row(r, _):
        a = attrv[pl.ds(j * C + r, 16)][0]
        for u in range(D // 16):
          sl = pl.ds(u * 16, 16)
          gbuf[b][r, sl] = gbuf[b][r, sl] * a
        return 0

      lax.fori_loop(0, C, row, 0)

    goff = jnp.where(cid == 0, sid * NG0, NS * NG0 + sid * NG1)

    def group(g, _):
      pltpu.sync_copy(srch.at[goff + g], srcv)
      pltpu.sync_copy(dsth.at[goff + g], dstv)
      pltpu.sync_copy(attrh.at[goff + g], attrv)

      live_g = {}
      live_s = {}
      for j in range(min(2, G)):
        live_g[j % NBUF] = pltpu.async_copy(
            h0.at[srcv.at[j]], gbuf[j % NBUF], gsem[j % NBUF])
      for j in range(G):
        b = j % NBUF
        live_g.pop(b).wait()
        scale(j, b)
        live_s[b] = pltpu.async_copy(
            gbuf[b], acc.at[dstv.at[j]], ssem[b], add=True)
        if j + 2 < G:
          nb = (j + 2) % NBUF
          if nb in live_s:
            live_s.pop(nb).wait()
          live_g[nb] = pltpu.async_copy(
              h0.at[srcv.at[j + 2]], gbuf[nb], gsem[nb])
      for b in sorted(live_s):
        live_s[b].wait()
      return 0

    lax.fori_loop(0, jnp.where(cid == 0, NG0, NG1), group, 0)
    plsc.subcore_barrier()

    rows = pl.ds(base, RPT)

    @pl.when(cid == 0)
    def _():
      pltpu.sync_copy(acc.at[rows], o0.at[rows])

    @pl.when(cid == 1)
    def _():
      pltpu.sync_copy(acc.at[rows], o1.at[rows])

  return shift


_shift = _make_shift()

_BM = 1000


def _mm_body(x_ref, a1, a2, a3, b3, w0, w1, w2, w3, bs, o_ref):
  acc = jnp.dot(x_ref[...], w0[...], preferred_element_type=jnp.float32)
  acc += jnp.dot(a1[...], w1[...], preferred_element_type=jnp.float32)
  acc += jnp.dot(a2[...], w2[...], preferred_element_type=jnp.float32)
  acc += jnp.dot(a3[...] + b3[...], w3[...], preferred_element_type=jnp.float32)
  o_ref[...] = acc + bs[0:1, :]


def _taps_matmul(x, h1, h2, h3a, h3b, W0, W1, W2, W3, bsum):
  hspec = pl.BlockSpec((_BM, D), lambda i: (i, 0))
  wspec = pl.BlockSpec((D, D), lambda i: (0, 0))
  bspec = pl.BlockSpec((8, D), lambda i: (0, 0))
  return pl.pallas_call(
      _mm_body,
      grid=(N // _BM,),
      in_specs=[hspec] * 5 + [wspec] * 4 + [bspec],
      out_specs=hspec,
      out_shape=jax.ShapeDtypeStruct((N, D), jnp.float32),
  )(x, h1, h2, h3a, h3b, W0, W1, W2, W3, bsum)


def _merge_body(a_ref, b_ref, o_ref):
  o_ref[...] = a_ref[...] + b_ref[...]


def _merge(a, b):
  bm = NPAD // 8
  spec = pl.BlockSpec((bm, D), lambda i: (i, 0))
  return pl.pallas_call(
      _merge_body,
      grid=(NPAD // bm,),
      in_specs=[spec, spec],
      out_specs=spec,
      out_shape=jax.ShapeDtypeStruct((NPAD, D), jnp.float32),
  )(a, b)


def kernel(x, edge_index, edge_attr, W0, W1, W2, W3, b0, b1, b2, b3):
  pad = EPAD - E
  shp = (NGTOT, G, C)
  src = jnp.concatenate(
      [edge_index[0], jnp.zeros((pad,), jnp.int32)]).reshape(shp)
  dst = jnp.concatenate(
      [edge_index[1], jnp.zeros((pad,), jnp.int32)]).reshape(shp)
  attr = jnp.concatenate(
      [edge_attr, jnp.zeros((pad,), jnp.float32)]).reshape(NGTOT, GEDGES)
  attr = jnp.pad(attr, ((0, 0), (0, 16)))

  h1a, h1b = _shift(x, src, dst, attr)
  h1 = _merge(h1a, h1b)
  h2a, h2b = _shift(h1, src, dst, attr)
  h2 = _merge(h2a, h2b)
  h3a, h3b = _shift(h2, src, dst, attr)

  bsum = jnp.broadcast_to((b0 + b1 + b2 + b3)[None, :], (8, D))
  return _taps_matmul(x, h1, h2, h3a, h3b, W0, W1, W2, W3, bsum)


# speedup vs baseline: 1.0988x; 1.0988x over previous
import functools

import numpy as np

import jax
import jax.numpy as jnp
from jax import lax
from jax.experimental import pallas as pl
from jax.experimental.pallas import tpu as pltpu
from jax.experimental.pallas import tpu_sc as plsc

N = 10000
D = 128
E = 320000
NC = 2
NS = 16
NW = NC * NS
C = 96
G = 21
NG0 = 6
NG1 = 4
GEDGES = G * C
NGTOT = NS * (NG0 + NG1)
EPAD = NGTOT * GEDGES
RPT = 632
NPAD = NS * RPT
NBUF = 3


def _make_shift():
  mesh = plsc.VectorSubcoreMesh(
      core_axis_name="c", subcore_axis_name="s", num_cores=NC, num_subcores=NS)
  out_type = (
      jax.ShapeDtypeStruct((NPAD, D), jnp.float32),
      jax.ShapeDtypeStruct((NPAD, D), jnp.float32),
  )
  scratch = (
      [
          pltpu.VMEM((G, C), jnp.int32),
          pltpu.VMEM((G, C), jnp.int32),
          pltpu.VMEM((G * C + 16,), jnp.float32),
      ]
      + [pltpu.VMEM((C, D), jnp.float32)] * NBUF
      + [pltpu.VMEM_SHARED((NPAD, D), jnp.float32)]
      + [pltpu.SemaphoreType.DMA] * (2 * NBUF)
  )

  @functools.partial(
      pl.kernel, out_type=out_type, mesh=mesh, scratch_types=scratch)
  def shift(h0, srch, dsth, attrh, o0, o1, srcv, dstv, attrv, *bufs_sems):
    gbuf = bufs_sems[:NBUF]
    acc = bufs_sems[NBUF]
    gsem = bufs_sems[NBUF + 1:NBUF + 1 + NBUF]
    ssem = bufs_sems[NBUF + 1 + NBUF:]
    cid = lax.axis_index("c")
    sid = lax.axis_index("s")
    wid = sid * NC + cid

    zero = jnp.zeros((16,), jnp.float32)

    def zrow(r, _):
      for u in range(D // 16):
        gbuf[0][r, pl.ds(u * 16, 16)] = zero
      return 0

    lax.fori_loop(0, C, zrow, 0)
    base = sid * RPT
    nfull, rem = divmod(RPT, C)
    for t in range(nfull):
      pltpu.sync_copy(gbuf[0], acc.at[pl.ds(base + t * C, C)])
    if rem:
      pltpu.sync_copy(gbuf[0].at[pl.ds(0, rem)],
                      acc.at[pl.ds(base + nfull * C, rem)])
    plsc.subcore_barrier()

    def scale(j, b):
      def row(r, _):
        a = attrv[pl.ds(j * C + r, 16)][0]
        for u in range(D // 16):
          sl = pl.ds(u * 16, 16)
          gbuf[b][r, sl] = gbuf[b][r, sl] * a
        return 0

      lax.fori_loop(0, C, row, 0)

    goff = jnp.where(cid == 0, sid * NG0, NS * NG0 + sid * NG1)

    def group(g, _):
      pltpu.sync_copy(srch.at[goff + g], srcv)
      pltpu.sync_copy(dsth.at[goff + g], dstv)
      pltpu.sync_copy(attrh.at[goff + g], attrv)

      live_g = {}
      live_s = {}
      for j in range(min(2, G)):
        live_g[j % NBUF] = pltpu.async_copy(
            h0.at[srcv.at[j]], gbuf[j % NBUF], gsem[j % NBUF])
      for j in range(G):
        b = j % NBUF
        live_g.pop(b).wait()
        scale(j, b)
        live_s[b] = pltpu.async_copy(
            gbuf[b], acc.at[dstv.at[j]], ssem[b], add=True)
        if j + 2 < G:
          nb = (j + 2) % NBUF
          if nb in live_s:
            live_s.pop(nb).wait()
          live_g[nb] = pltpu.async_copy(
              h0.at[srcv.at[j + 2]], gbuf[nb], gsem[nb])
      for b in sorted(live_s):
        live_s[b].wait()
      return 0

    lax.fori_loop(0, jnp.where(cid == 0, NG0, NG1), group, 0)
    plsc.subcore_barrier()

    rows = pl.ds(base, RPT)

    @pl.when(cid == 0)
    def _():
      pltpu.sync_copy(acc.at[rows], o0.at[rows])

    @pl.when(cid == 1)
    def _():
      pltpu.sync_copy(acc.at[rows], o1.at[rows])

  return shift


_shift = _make_shift()

_BM = 1000


def _mm_body(x_ref, a1, a2, a3, b3, w0, w1, w2, w3, bs, o_ref):
  acc = jnp.dot(x_ref[...], w0[...], preferred_element_type=jnp.float32)
  acc += jnp.dot(a1[...], w1[...], preferred_element_type=jnp.float32)
  acc += jnp.dot(a2[...], w2[...], preferred_element_type=jnp.float32)
  acc += jnp.dot(a3[...] + b3[...], w3[...], preferred_element_type=jnp.float32)
  o_ref[...] = acc + bs[0:1, :]


def _taps_matmul(x, h1, h2, h3a, h3b, W0, W1, W2, W3, bsum):
  hspec = pl.BlockSpec((_BM, D), lambda i: (i, 0))
  wspec = pl.BlockSpec((D, D), lambda i: (0, 0))
  bspec = pl.BlockSpec((8, D), lambda i: (0, 0))
  return pl.pallas_call(
      _mm_body,
      grid=(N // _BM,),
      in_specs=[hspec] * 5 + [wspec] * 4 + [bspec],
      out_specs=hspec,
      out_shape=jax.ShapeDtypeStruct((N, D), jnp.float32),
  )(x, h1, h2, h3a, h3b, W0, W1, W2, W3, bsum)


def _merge_body(a_ref, b_ref, o_ref):
  o_ref[...] = a_ref[...] + b_ref[...]


def _merge(a, b):
  bm = NPAD // 8
  spec = pl.BlockSpec((bm, D), lambda i: (i, 0))
  return pl.pallas_call(
      _merge_body,
      grid=(NPAD // bm,),
      in_specs=[spec, spec],
      out_specs=spec,
      out_shape=jax.ShapeDtypeStruct((NPAD, D), jnp.float32),
  )(a, b)


def kernel(x, edge_index, edge_attr, W0, W1, W2, W3, b0, b1, b2, b3):
  pad = EPAD - E
  shp = (NGTOT, G, C)
  src = jnp.concatenate(
      [edge_index[0], jnp.zeros((pad,), jnp.int32)]).reshape(shp)
  dst = jnp.concatenate(
      [edge_index[1], jnp.zeros((pad,), jnp.int32)]).reshape(shp)
  attr = jnp.concatenate(
      [edge_attr, jnp.zeros((pad,), jnp.float32)]).reshape(NGTOT, GEDGES)
  attr = jnp.pad(attr, ((0, 0), (0, 16)))

  h1a, h1b = _shift(x, src, dst, attr)
  h1 = _merge(h1a, h1b)
  h2a, h2b = _shift(h1, src, dst, attr)
  h2 = _merge(h2a, h2b)
  h3a, h3b = _shift(h2, src, dst, attr)

  bsum = jnp.broadcast_to((b0 + b1 + b2 + b3)[None, :], (8, D))
  return _taps_matmul(x, h1, h2, h3a, h3b, W0, W1, W2, W3, bsum)


# speedup vs baseline: 1.1294x; 1.0278x over previous
import functools

import numpy as np

import jax
import jax.numpy as jnp
from jax import lax
from jax.experimental import pallas as pl
from jax.experimental.pallas import tpu as pltpu
from jax.experimental.pallas import tpu_sc as plsc

N = 10000
D = 128
E = 320000
NC = 2
NS = 16
NW = NC * NS
C = 96
G = 21
NG0 = 8
NG1 = 2
GEDGES = G * C
NGTOT = NS * (NG0 + NG1)
EPAD = NGTOT * GEDGES
RPT = 632
NPAD = NS * RPT
NBUF = 3


def _make_shift():
  mesh = plsc.VectorSubcoreMesh(
      core_axis_name="c", subcore_axis_name="s", num_cores=NC, num_subcores=NS)
  out_type = (
      jax.ShapeDtypeStruct((NPAD, D), jnp.float32),
      jax.ShapeDtypeStruct((NPAD, D), jnp.float32),
  )
  scratch = (
      [
          pltpu.VMEM((G, C), jnp.int32),
          pltpu.VMEM((G, C), jnp.int32),
          pltpu.VMEM((G * C + 16,), jnp.float32),
      ]
      + [pltpu.VMEM((C, D), jnp.float32)] * NBUF
      + [pltpu.VMEM_SHARED((NPAD, D), jnp.float32)]
      + [pltpu.SemaphoreType.DMA] * (2 * NBUF)
  )

  @functools.partial(
      pl.kernel, out_type=out_type, mesh=mesh, scratch_types=scratch)
  def shift(h0, srch, dsth, attrh, o0, o1, srcv, dstv, attrv, *bufs_sems):
    gbuf = bufs_sems[:NBUF]
    acc = bufs_sems[NBUF]
    gsem = bufs_sems[NBUF + 1:NBUF + 1 + NBUF]
    ssem = bufs_sems[NBUF + 1 + NBUF:]
    cid = lax.axis_index("c")
    sid = lax.axis_index("s")
    wid = sid * NC + cid

    zero = jnp.zeros((16,), jnp.float32)

    def zrow(r, _):
      for u in range(D // 16):
        gbuf[0][r, pl.ds(u * 16, 16)] = zero
      return 0

    lax.fori_loop(0, C, zrow, 0)
    base = sid * RPT
    nfull, rem = divmod(RPT, C)
    for t in range(nfull):
      pltpu.sync_copy(gbuf[0], acc.at[pl.ds(base + t * C, C)])
    if rem:
      pltpu.sync_copy(gbuf[0].at[pl.ds(0, rem)],
                      acc.at[pl.ds(base + nfull * C, rem)])
    plsc.subcore_barrier()

    def scale(j, b):
      def row(r, _):
        a = attrv[pl.ds(j * C + r, 16)][0]
        for u in range(D // 16):
          sl = pl.ds(u * 16, 16)
          gbuf[b][r, sl] = gbuf[b][r, sl] * a
        return 0

      lax.fori_loop(0, C, row, 0)

    goff = jnp.where(cid == 0, sid * NG0, NS * NG0 + sid * NG1)

    def group(g, _):
      pltpu.sync_copy(srch.at[goff + g], srcv)
      pltpu.sync_copy(dsth.at[goff + g], dstv)
      pltpu.sync_copy(attrh.at[goff + g], attrv)

      live_g = {}
      live_s = {}
      for j in range(min(2, G)):
        live_g[j % NBUF] = pltpu.async_copy(
            h0.at[srcv.at[j]], gbuf[j % NBUF], gsem[j % NBUF])
      for j in range(G):
        b = j % NBUF
        live_g.pop(b).wait()
        scale(j, b)
        live_s[b] = pltpu.async_copy(
            gbuf[b], acc.at[dstv.at[j]], ssem[b], add=True)
        if j + 2 < G:
          nb = (j + 2) % NBUF
          if nb in live_s:
            live_s.pop(nb).wait()
          live_g[nb] = pltpu.async_copy(
              h0.at[srcv.at[j + 2]], gbuf[nb], gsem[nb])
      for b in sorted(live_s):
        live_s[b].wait()
      return 0

    lax.fori_loop(0, jnp.where(cid == 0, NG0, NG1), group, 0)
    plsc.subcore_barrier()

    rows = pl.ds(base, RPT)

    @pl.when(cid == 0)
    def _():
      pltpu.sync_copy(acc.at[rows], o0.at[rows])

    @pl.when(cid == 1)
    def _():
      pltpu.sync_copy(acc.at[rows], o1.at[rows])

  return shift


_shift = _make_shift()

_BM = 1000


def _mm_body(x_ref, a1, a2, a3, b3, w0, w1, w2, w3, bs, o_ref):
  acc = jnp.dot(x_ref[...], w0[...], preferred_element_type=jnp.float32)
  acc += jnp.dot(a1[...], w1[...], preferred_element_type=jnp.float32)
  acc += jnp.dot(a2[...], w2[...], preferred_element_type=jnp.float32)
  acc += jnp.dot(a3[...] + b3[...], w3[...], preferred_element_type=jnp.float32)
  o_ref[...] = acc + bs[0:1, :]


def _taps_matmul(x, h1, h2, h3a, h3b, W0, W1, W2, W3, bsum):
  hspec = pl.BlockSpec((_BM, D), lambda i: (i, 0))
  wspec = pl.BlockSpec((D, D), lambda i: (0, 0))
  bspec = pl.BlockSpec((8, D), lambda i: (0, 0))
  return pl.pallas_call(
      _mm_body,
      grid=(N // _BM,),
      in_specs=[hspec] * 5 + [wspec] * 4 + [bspec],
      out_specs=hspec,
      out_shape=jax.ShapeDtypeStruct((N, D), jnp.float32),
  )(x, h1, h2, h3a, h3b, W0, W1, W2, W3, bsum)


def _merge_body(a_ref, b_ref, o_ref):
  o_ref[...] = a_ref[...] + b_ref[...]


def _merge(a, b):
  bm = NPAD // 8
  spec = pl.BlockSpec((bm, D), lambda i: (i, 0))
  return pl.pallas_call(
      _merge_body,
      grid=(NPAD // bm,),
      in_specs=[spec, spec],
      out_specs=spec,
      out_shape=jax.ShapeDtypeStruct((NPAD, D), jnp.float32),
  )(a, b)


def kernel(x, edge_index, edge_attr, W0, W1, W2, W3, b0, b1, b2, b3):
  pad = EPAD - E
  shp = (NGTOT, G, C)
  src = jnp.concatenate(
      [edge_index[0], jnp.zeros((pad,), jnp.int32)]).reshape(shp)
  dst = jnp.concatenate(
      [edge_index[1], jnp.zeros((pad,), jnp.int32)]).reshape(shp)
  attr = jnp.concatenate(
      [edge_attr, jnp.zeros((pad,), jnp.float32)]).reshape(NGTOT, GEDGES)
  attr = jnp.pad(attr, ((0, 0), (0, 16)))

  h1a, h1b = _shift(x, src, dst, attr)
  h1 = _merge(h1a, h1b)
  h2a, h2b = _shift(h1, src, dst, attr)
  h2 = _merge(h2a, h2b)
  h3a, h3b = _shift(h2, src, dst, attr)

  bsum = jnp.broadcast_to((b0 + b1 + b2 + b3)[None, :], (8, D))
  return _taps_matmul(x, h1, h2, h3a, h3b, W0, W1, W2, W3, bsum)


# speedup vs baseline: 1.3298x; 1.1775x over previous
import functools

import numpy as np

import jax
import jax.numpy as jnp
from jax import lax
from jax.experimental import pallas as pl
from jax.experimental.pallas import tpu as pltpu
from jax.experimental.pallas import tpu_sc as plsc

N = 10000
D = 128
E = 320000
NC = 2
NS = 16
NW = NC * NS
C = 72
G = 28
NG0 = 7
NG1 = 3
GEDGES = G * C
NGTOT = NS * (NG0 + NG1)
EPAD = NGTOT * GEDGES
RPT = 632
NPAD = NS * RPT
NBUF = 4


def _make_shift():
  mesh = plsc.VectorSubcoreMesh(
      core_axis_name="c", subcore_axis_name="s", num_cores=NC, num_subcores=NS)
  out_type = (
      jax.ShapeDtypeStruct((NPAD, D), jnp.float32),
      jax.ShapeDtypeStruct((NPAD, D), jnp.float32),
  )
  scratch = (
      [
          pltpu.VMEM((G, C), jnp.int32),
          pltpu.VMEM((G, C), jnp.int32),
          pltpu.VMEM((G * C + 16,), jnp.float32),
      ]
      + [pltpu.VMEM((C, D), jnp.float32)] * NBUF
      + [pltpu.VMEM_SHARED((NPAD, D), jnp.float32)]
      + [pltpu.SemaphoreType.DMA] * (2 * NBUF)
  )

  @functools.partial(
      pl.kernel, out_type=out_type, mesh=mesh, scratch_types=scratch)
  def shift(h0, srch, dsth, attrh, o0, o1, srcv, dstv, attrv, *bufs_sems):
    gbuf = bufs_sems[:NBUF]
    acc = bufs_sems[NBUF]
    gsem = bufs_sems[NBUF + 1:NBUF + 1 + NBUF]
    ssem = bufs_sems[NBUF + 1 + NBUF:]
    cid = lax.axis_index("c")
    sid = lax.axis_index("s")
    wid = sid * NC + cid

    zero = jnp.zeros((16,), jnp.float32)

    def zrow(r, _):
      for u in range(D // 16):
        gbuf[0][r, pl.ds(u * 16, 16)] = zero
      return 0

    lax.fori_loop(0, C, zrow, 0)
    base = sid * RPT
    nfull, rem = divmod(RPT, C)
    for t in range(nfull):
      pltpu.sync_copy(gbuf[0], acc.at[pl.ds(base + t * C, C)])
    if rem:
      pltpu.sync_copy(gbuf[0].at[pl.ds(0, rem)],
                      acc.at[pl.ds(base + nfull * C, rem)])
    plsc.subcore_barrier()

    def scale(j, b):
      def row(r, _):
        a = attrv[pl.ds(j * C + r, 16)][0]
        for u in range(D // 16):
          sl = pl.ds(u * 16, 16)
          gbuf[b][r, sl] = gbuf[b][r, sl] * a
        return 0

      lax.fori_loop(0, C, row, 0)

    goff = jnp.where(cid == 0, sid * NG0, NS * NG0 + sid * NG1)

    def group(g, _):
      pltpu.sync_copy(srch.at[goff + g], srcv)
      pltpu.sync_copy(dsth.at[goff + g], dstv)
      pltpu.sync_copy(attrh.at[goff + g], attrv)

      live_g = {}
      live_s = {}
      for j in range(min(3, G)):
        live_g[j % NBUF] = pltpu.async_copy(
            h0.at[srcv.at[j]], gbuf[j % NBUF], gsem[j % NBUF])
      for j in range(G):
        b = j % NBUF
        live_g.pop(b).wait()
        scale(j, b)
        live_s[b] = pltpu.async_copy(
            gbuf[b], acc.at[dstv.at[j]], ssem[b], add=True)
        if j + 3 < G:
          nb = (j + 3) % NBUF
          if nb in live_s:
            live_s.pop(nb).wait()
          live_g[nb] = pltpu.async_copy(
              h0.at[srcv.at[j + 3]], gbuf[nb], gsem[nb])
      for b in sorted(live_s):
        live_s[b].wait()
      return 0

    lax.fori_loop(0, jnp.where(cid == 0, NG0, NG1), group, 0)
    plsc.subcore_barrier()

    rows = pl.ds(base, RPT)

    @pl.when(cid == 0)
    def _():
      pltpu.sync_copy(acc.at[rows], o0.at[rows])

    @pl.when(cid == 1)
    def _():
      pltpu.sync_copy(acc.at[rows], o1.at[rows])

  return shift


_shift = _make_shift()

_BM = 1000


def _mm_body(x_ref, a1, a2, a3, b3, w0, w1, w2, w3, bs, o_ref):
  acc = jnp.dot(x_ref[...], w0[...], preferred_element_type=jnp.float32)
  acc += jnp.dot(a1[...], w1[...], preferred_element_type=jnp.float32)
  acc += jnp.dot(a2[...], w2[...], preferred_element_type=jnp.float32)
  acc += jnp.dot(a3[...] + b3[...], w3[...], preferred_element_type=jnp.float32)
  o_ref[...] = acc + bs[0:1, :]


def _taps_matmul(x, h1, h2, h3a, h3b, W0, W1, W2, W3, bsum):
  hspec = pl.BlockSpec((_BM, D), lambda i: (i, 0))
  wspec = pl.BlockSpec((D, D), lambda i: (0, 0))
  bspec = pl.BlockSpec((8, D), lambda i: (0, 0))
  return pl.pallas_call(
      _mm_body,
      grid=(N // _BM,),
      in_specs=[hspec] * 5 + [wspec] * 4 + [bspec],
      out_specs=hspec,
      out_shape=jax.ShapeDtypeStruct((N, D), jnp.float32),
  )(x, h1, h2, h3a, h3b, W0, W1, W2, W3, bsum)


def _merge_body(a_ref, b_ref, o_ref):
  o_ref[...] = a_ref[...] + b_ref[...]


def _merge(a, b):
  bm = NPAD // 8
  spec = pl.BlockSpec((bm, D), lambda i: (i, 0))
  return pl.pallas_call(
      _merge_body,
      grid=(NPAD // bm,),
      in_specs=[spec, spec],
      out_specs=spec,
      out_shape=jax.ShapeDtypeStruct((NPAD, D), jnp.float32),
  )(a, b)


def kernel(x, edge_index, edge_attr, W0, W1, W2, W3, b0, b1, b2, b3):
  pad = EPAD - E
  shp = (NGTOT, G, C)
  src = jnp.concatenate(
      [edge_index[0], jnp.zeros((pad,), jnp.int32)]).reshape(shp)
  dst = jnp.concatenate(
      [edge_index[1], jnp.zeros((pad,), jnp.int32)]).reshape(shp)
  attr = jnp.concatenate(
      [edge_attr, jnp.zeros((pad,), jnp.float32)]).reshape(NGTOT, GEDGES)
  attr = jnp.pad(attr, ((0, 0), (0, 16)))

  h1a, h1b = _shift(x, src, dst, attr)
  h1 = _merge(h1a, h1b)
  h2a, h2b = _shift(h1, src, dst, attr)
  h2 = _merge(h2a, h2b)
  h3a, h3b = _shift(h2, src, dst, attr)

  bsum = jnp.broadcast_to((b0 + b1 + b2 + b3)[None, :], (8, D))
  return _taps_matmul(x, h1, h2, h3a, h3b, W0, W1, W2, W3, bsum)


# speedup vs baseline: 1.3500x; 1.0152x over previous
import functools

import numpy as np

import jax
import jax.numpy as jnp
from jax import lax
from jax.experimental import pallas as pl
from jax.experimental.pallas import tpu as pltpu
from jax.experimental.pallas import tpu_sc as plsc

N = 10000
D = 128
E = 320000
NC = 2
NS = 16
NW = NC * NS
C = 56
G = 36
NG0 = 7
NG1 = 3
GEDGES = G * C
NGTOT = NS * (NG0 + NG1)
EPAD = NGTOT * GEDGES
RPT = 632
NPAD = NS * RPT
NBUF = 5


def _make_shift():
  mesh = plsc.VectorSubcoreMesh(
      core_axis_name="c", subcore_axis_name="s", num_cores=NC, num_subcores=NS)
  out_type = (
      jax.ShapeDtypeStruct((NPAD, D), jnp.float32),
      jax.ShapeDtypeStruct((NPAD, D), jnp.float32),
  )
  scratch = (
      [
          pltpu.VMEM((G, C), jnp.int32),
          pltpu.VMEM((G, C), jnp.int32),
          pltpu.VMEM((G * C + 16,), jnp.float32),
      ]
      + [pltpu.VMEM((C, D), jnp.float32)] * NBUF
      + [pltpu.VMEM_SHARED((NPAD, D), jnp.float32)]
      + [pltpu.SemaphoreType.DMA] * (2 * NBUF)
  )

  @functools.partial(
      pl.kernel, out_type=out_type, mesh=mesh, scratch_types=scratch)
  def shift(h0, srch, dsth, attrh, o0, o1, srcv, dstv, attrv, *bufs_sems):
    gbuf = bufs_sems[:NBUF]
    acc = bufs_sems[NBUF]
    gsem = bufs_sems[NBUF + 1:NBUF + 1 + NBUF]
    ssem = bufs_sems[NBUF + 1 + NBUF:]
    cid = lax.axis_index("c")
    sid = lax.axis_index("s")
    wid = sid * NC + cid

    zero = jnp.zeros((16,), jnp.float32)

    def zrow(r, _):
      for u in range(D // 16):
        gbuf[0][r, pl.ds(u * 16, 16)] = zero
      return 0

    lax.fori_loop(0, C, zrow, 0)
    base = sid * RPT
    nfull, rem = divmod(RPT, C)
    for t in range(nfull):
      pltpu.sync_copy(gbuf[0], acc.at[pl.ds(base + t * C, C)])
    if rem:
      pltpu.sync_copy(gbuf[0].at[pl.ds(0, rem)],
                      acc.at[pl.ds(base + nfull * C, rem)])
    plsc.subcore_barrier()

    def scale(j, b):
      def row(r, _):
        a = attrv[pl.ds(j * C + r, 16)][0]
        for u in range(D // 16):
          sl = pl.ds(u * 16, 16)
          gbuf[b][r, sl] = gbuf[b][r, sl] * a
        return 0

      lax.fori_loop(0, C, row, 0)

    goff = jnp.where(cid == 0, sid * NG0, NS * NG0 + sid * NG1)

    def group(g, _):
      pltpu.sync_copy(srch.at[goff + g], srcv)
      pltpu.sync_copy(dsth.at[goff + g], dstv)
      pltpu.sync_copy(attrh.at[goff + g], attrv)

      live_g = {}
      live_s = {}
      for j in range(min(4, G)):
        live_g[j % NBUF] = pltpu.async_copy(
            h0.at[srcv.at[j]], gbuf[j % NBUF], gsem[j % NBUF])
      for j in range(G):
        b = j % NBUF
        live_g.pop(b).wait()
        scale(j, b)
        live_s[b] = pltpu.async_copy(
            gbuf[b], acc.at[dstv.at[j]], ssem[b], add=True)
        if j + 4 < G:
          nb = (j + 4) % NBUF
          if nb in live_s:
            live_s.pop(nb).wait()
          live_g[nb] = pltpu.async_copy(
              h0.at[srcv.at[j + 4]], gbuf[nb], gsem[nb])
      for b in sorted(live_s):
        live_s[b].wait()
      return 0

    lax.fori_loop(0, jnp.where(cid == 0, NG0, NG1), group, 0)
    plsc.subcore_barrier()

    rows = pl.ds(base, RPT)

    @pl.when(cid == 0)
    def _():
      pltpu.sync_copy(acc.at[rows], o0.at[rows])

    @pl.when(cid == 1)
    def _():
      pltpu.sync_copy(acc.at[rows], o1.at[rows])

  return shift


_shift = _make_shift()

_BM = 1000


def _mm_body(x_ref, a1, a2, a3, b3, w0, w1, w2, w3, bs, o_ref):
  acc = jnp.dot(x_ref[...], w0[...], preferred_element_type=jnp.float32)
  acc += jnp.dot(a1[...], w1[...], preferred_element_type=jnp.float32)
  acc += jnp.dot(a2[...], w2[...], preferred_element_type=jnp.float32)
  acc += jnp.dot(a3[...] + b3[...], w3[...], preferred_element_type=jnp.float32)
  o_ref[...] = acc + bs[0:1, :]


def _taps_matmul(x, h1, h2, h3a, h3b, W0, W1, W2, W3, bsum):
  hspec = pl.BlockSpec((_BM, D), lambda i: (i, 0))
  wspec = pl.BlockSpec((D, D), lambda i: (0, 0))
  bspec = pl.BlockSpec((8, D), lambda i: (0, 0))
  return pl.pallas_call(
      _mm_body,
      grid=(N // _BM,),
      in_specs=[hspec] * 5 + [wspec] * 4 + [bspec],
      out_specs=hspec,
      out_shape=jax.ShapeDtypeStruct((N, D), jnp.float32),
  )(x, h1, h2, h3a, h3b, W0, W1, W2, W3, bsum)


def _merge_body(a_ref, b_ref, o_ref):
  o_ref[...] = a_ref[...] + b_ref[...]


def _merge(a, b):
  bm = NPAD // 8
  spec = pl.BlockSpec((bm, D), lambda i: (i, 0))
  return pl.pallas_call(
      _merge_body,
      grid=(NPAD // bm,),
      in_specs=[spec, spec],
      out_specs=spec,
      out_shape=jax.ShapeDtypeStruct((NPAD, D), jnp.float32),
  )(a, b)


def kernel(x, edge_index, edge_attr, W0, W1, W2, W3, b0, b1, b2, b3):
  pad = EPAD - E
  shp = (NGTOT, G, C)
  src = jnp.concatenate(
      [edge_index[0], jnp.zeros((pad,), jnp.int32)]).reshape(shp)
  dst = jnp.concatenate(
      [edge_index[1], jnp.zeros((pad,), jnp.int32)]).reshape(shp)
  attr = jnp.concatenate(
      [edge_attr, jnp.zeros((pad,), jnp.float32)]).reshape(NGTOT, GEDGES)
  attr = jnp.pad(attr, ((0, 0), (0, 16)))

  h1a, h1b = _shift(x, src, dst, attr)
  h1 = _merge(h1a, h1b)
  h2a, h2b = _shift(h1, src, dst, attr)
  h2 = _merge(h2a, h2b)
  h3a, h3b = _shift(h2, src, dst, attr)

  bsum = jnp.broadcast_to((b0 + b1 + b2 + b3)[None, :], (8, D))
  return _taps_matmul(x, h1, h2, h3a, h3b, W0, W1, W2, W3, bsum)
